# baseline (speedup 1.0000x reference)
import functools

import jax
import jax.numpy as jnp
from jax import lax
from jax.experimental import pallas as pl
from jax.experimental.pallas import tpu as pltpu
from jax.experimental.pallas import tpu_sc as plsc

NC = 2
NS = 16
HALF = 128


def _chunk(epw):
    for k in range(128, 0, -8):
        if epw % k == 0:
            return k
    raise ValueError(epw)


def _row_copy(sid, src, dst, n):
    rpw = (n // NS) // 8 * 8
    tail = n - NS * rpw
    r0 = sid * rpw
    pltpu.sync_copy(src.at[pl.ds(r0, rpw)], dst.at[pl.ds(r0, rpw)])
    if tail:
        @pl.when(sid == NS - 1)
        def _():
            pltpu.sync_copy(src.at[pl.ds(NS * rpw, tail)],
                            dst.at[pl.ds(NS * rpw, tail)])


def _build_agg_wide(n, e):
    assert e % NS == 0 and n % NS == 0
    epw = e // NS
    k = 128
    nchunks = epw // k
    tail = epw - nchunks * k
    assert tail % 8 == 0 and nchunks >= 8
    mesh = plsc.VectorSubcoreMesh(
        core_axis_name="c", subcore_axis_name="s", num_cores=NC, num_subcores=NS
    )

    NBD = 3
    NBC = 3
    NBS = 1
    scratch = ([pltpu.VMEM((k,), jnp.int32)] * NBC
               + [pltpu.VMEM((k,), jnp.int32)] * NBC
               + [pltpu.VMEM((k, HALF), jnp.float32)] * NBD
               + ([pltpu.VMEM((tail,), jnp.int32)] if tail else [])
               + [pltpu.VMEM_SHARED((n, HALF), jnp.float32)]
               + [pltpu.SemaphoreType.DMA] * (2 * NBC + NBD + NBS + (1 if tail else 0)))

    @functools.partial(
        pl.kernel,
        out_type=(
            jax.ShapeDtypeStruct((n, HALF), jnp.float32),
            jax.ShapeDtypeStruct((n, HALF), jnp.float32),
        ),
        mesh=mesh,
        scratch_types=scratch,
    )
    def kern(lo_hbm, hi_hbm, row_hbm, col_hbm, out_lo, out_hi, *scr):
        rows = scr[0:NBC]
        cols = scr[NBC:2 * NBC]
        bufs = scr[2 * NBC:2 * NBC + NBD]
        p = 2 * NBC + NBD
        if tail:
            tcol = scr[p]
            p += 1
        acc_sh = scr[p]
        sems = scr[p + 1:]
        semr = sems[0:NBC]
        semc = sems[NBC:2 * NBC]
        semg = sems[2 * NBC:2 * NBC + NBD]
        sems_ = sems[2 * NBC + NBD:2 * NBC + NBD + NBS]

        cid = lax.axis_index("c")
        sid = lax.axis_index("s")
        ebase = sid * epw

        def run(x_hbm, out_hbm):
            def issue_c(j, rc):
                off = ebase + j * k
                pltpu.async_copy(row_hbm.at[pl.ds(off, k)], rows[rc], semr[rc])
                pltpu.async_copy(col_hbm.at[pl.ds(off, k)], cols[rc], semc[rc])

            def wait_row(rc):
                pltpu.make_async_copy(row_hbm.at[pl.ds(0, k)],
                                      rows[rc], semr[rc]).wait()

            def wait_c(rc):
                pltpu.make_async_copy(col_hbm.at[pl.ds(0, k)],
                                      cols[rc], semc[rc]).wait()

            def issue_g(j, rd):
                pltpu.async_copy(x_hbm.at[rows[rd % NBC]], bufs[rd], semg[rd])

            def wait_g(rd):
                pltpu.make_async_copy(x_hbm.at[rows[rd % NBC]],
                                      bufs[rd], semg[rd]).wait()

            def issue_s(rd, rc, rs):
                pltpu.async_copy(bufs[rd], acc_sh.at[cols[rc]], sems_[rs],
                                 add=True)

            def wait_s(rd, rc, rs):
                pltpu.make_async_copy(bufs[rd], acc_sh.at[cols[rc]],
                                      sems_[rs]).wait()

            def body(j, q, first=False, has_c=True, has_g=True):
                rd, rc, rs = q % NBD, q % NBC, q % NBS
                if not first:
                    wait_s((q - 1) % NBD, (q - 1) % NBC, (q - 1) % NBS)
                if has_c:
                    issue_c(j + 2, (q + 2) % NBC)
                wait_g(rd)
                wait_c(rc)
                issue_s(rd, rc, rs)
                if has_g:
                    wait_row((q + 1) % NBC)
                    issue_g(j + 1, (q + 1) % NBD)

            _row_copy(sid, x_hbm, acc_sh, n)
            plsc.subcore_barrier()

            issue_c(0, 0)
            issue_c(1, 1)
            wait_row(0)
            issue_g(0, 0)
            body(0, 0, first=True)
            body(1, 1)

            steady = ((nchunks - 4) // 3) * 3
            if steady > 0:
                def loop_body(u, _):
                    j0 = 2 + u * 3
                    for q in range(3):
                        body(j0 + q, 2 + q)
                    return 0

                lax.fori_loop(0, steady // 3, loop_body, 0)

            for j in range(2 + steady, nchunks):
                body(j, j, has_c=(j + 2 < nchunks), has_g=(j + 1 < nchunks))
            j = nchunks - 1
            wait_s(j % NBD, j % NBC, j % NBS)

            if tail:
                off = ebase + nchunks * k
                pltpu.sync_copy(row_hbm.at[pl.ds(off, tail)],
                                rows[0].at[pl.ds(0, tail)])
                pltpu.sync_copy(col_hbm.at[pl.ds(off, tail)], tcol)
                tb = bufs[0].at[pl.ds(0, tail)]
                pltpu.async_copy(x_hbm.at[rows[0].at[pl.ds(0, tail)]], tb,
                                 sems[2 * NBC + NBD + NBS]).wait()
                pltpu.sync_copy(tb, acc_sh.at[tcol], add=True)

            plsc.subcore_barrier()
            _row_copy(sid, acc_sh, out_hbm, n)

        @pl.when(cid == 0)
        def _():
            run(lo_hbm, out_lo)

        @pl.when(cid == 1)
        def _():
            run(hi_hbm, out_hi)

    return kern


def _build_agg_scalar(n, e):
    nw = NC * NS
    L = 16
    assert n % L == 0
    epw = -(-e // nw)
    epw_pad = -(-epw // L) * L
    assert (epw * (nw - 1)) % 8 == 0 and epw % 8 == 0
    nvec = epw // L
    rem = epw - nvec * L
    mesh = plsc.VectorSubcoreMesh(
        core_axis_name="c", subcore_axis_name="s", num_cores=NC, num_subcores=NS
    )

    @functools.partial(
        pl.kernel,
        out_type=jax.ShapeDtypeStruct((nw * n,), jnp.float32),
        mesh=mesh,
        compiler_params=pltpu.CompilerParams(needs_layout_passes=False),
        scratch_types=[
            pltpu.VMEM((n,), jnp.float32),
            pltpu.VMEM((n,), jnp.float32),
            pltpu.VMEM((epw_pad,), jnp.int32),
            pltpu.VMEM((epw_pad,), jnp.int32),
        ],
    )
    def kern(tab_hbm, row_hbm, col_hbm, out_hbm, tab_v, acc_v, row_v, col_v):
        cid = lax.axis_index("c")
        sid = lax.axis_index("s")
        wid = cid * NS + sid
        ebase = wid * epw

        zeros = jnp.zeros((L,), jnp.float32)

        def zbody(i, _):
            acc_v[pl.ds(i * L, L)] = zeros
            return 0

        lax.fori_loop(0, n // L, zbody, 0)
        pltpu.sync_copy(tab_hbm, tab_v)
        if epw_pad > epw:
            col_v[pl.ds(epw_pad - L, L)] = jnp.zeros((L,), jnp.int32)
            row_v[pl.ds(epw_pad - L, L)] = jnp.zeros((L,), jnp.int32)
        pltpu.sync_copy(row_hbm.at[pl.ds(ebase, epw)], row_v.at[pl.ds(0, epw)])
        pltpu.sync_copy(col_hbm.at[pl.ds(ebase, epw)], col_v.at[pl.ds(0, epw)])

        def body(j, _):
            idx = row_v[pl.ds(j * L, L)]
            cols = col_v[pl.ds(j * L, L)]
            vals = plsc.load_gather(tab_v, [idx])
            plsc.addupdate_scatter(acc_v, [cols], vals)
            return 0

        lax.fori_loop(0, nvec, body, 0)
        if rem:
            mask = lax.iota(jnp.int32, L) < rem
            idx = row_v[pl.ds(nvec * L, L)]
            cols = col_v[pl.ds(nvec * L, L)]
            vals = plsc.load_gather(tab_v, [idx], mask=mask)
            plsc.addupdate_scatter(acc_v, [cols], vals, mask=mask)
        pltpu.sync_copy(acc_v, out_hbm.at[pl.ds(wid * n, n)])

    return kern


BLK = 2048


def _mm_first(x_ref, w_ref, dp_ref, lo_ref, hi_ref, dinv_ref):
    deg = jnp.sum(dp_ref[...], axis=0) + 1.0
    dinv = lax.rsqrt(deg)
    dinv_ref[...] = dinv
    xl = lax.dot_general(x_ref[...], w_ref[...], (((1,), (1,)), ((), ())),
                         preferred_element_type=jnp.float32)
    xs = xl * dinv[:, None]
    lo_ref[...] = xs[:, :HALF]
    hi_ref[...] = xs[:, HALF:]


def _mm_mid(lo_ref, hi_ref, dinv_ref, b_ref, w_ref, olo_ref, ohi_ref):
    dinv = dinv_ref[...][:, None]
    b = b_ref[...]
    h_lo = jnp.tanh(dinv * lo_ref[...] + b[:, :HALF])
    h_hi = jnp.tanh(dinv * hi_ref[...] + b[:, HALF:])
    w = w_ref[...]
    xl = (lax.dot_general(h_lo, w[:, :HALF], (((1,), (1,)), ((), ())),
                          preferred_element_type=jnp.float32)
          + lax.dot_general(h_hi, w[:, HALF:], (((1,), (1,)), ((), ())),
                            preferred_element_type=jnp.float32))
    xs = xl * dinv
    olo_ref[...] = xs[:, :HALF]
    ohi_ref[...] = xs[:, HALF:]


def _mm_last(lo_ref, hi_ref, dinv_ref, b_ref, w2_ref, s_ref):
    dinv = dinv_ref[...][:, None]
    b = b_ref[...]
    h_lo = jnp.tanh(dinv * lo_ref[...] + b[:, :HALF])
    h_hi = jnp.tanh(dinv * hi_ref[...] + b[:, HALF:])
    w2 = w2_ref[...]
    s = (lax.dot_general(h_lo, w2[:, :HALF], (((1,), (1,)), ((), ())),
                         preferred_element_type=jnp.float32)
         + lax.dot_general(h_hi, w2[:, HALF:], (((1,), (1,)), ((), ())),
                           preferred_element_type=jnp.float32))
    s_ref[...] = (s * dinv)[:, 0]


def _mm_fin(sp_ref, s_ref, dinv_ref, b2_ref, o_ref):
    agg = jnp.sum(sp_ref[...], axis=0) + s_ref[...]
    z = dinv_ref[...] * agg + b2_ref[...]
    o_ref[...] = 1.0 / (1.0 + jnp.exp(-z))


def kernel(x, edge_index, W0, b0, W1, b1, W2, b2):
    n, f_in = x.shape
    d = W0.shape[0]
    e = edge_index.shape[1]
    nw = NC * NS
    row = edge_index[0]
    col = edge_index[1]

    agg_wide = _build_agg_wide(n, e)
    agg_scalar = _build_agg_scalar(n, e)
    f32 = jnp.float32
    grid = (pl.cdiv(n, BLK),)

    def rblk(w):
        return pl.BlockSpec((BLK, w), lambda i: (i, 0))

    def full(a, b):
        return pl.BlockSpec((a, b), lambda i: (0, 0))

    vec = pl.BlockSpec((BLK,), lambda i: (i,))

    dp = agg_scalar(jnp.ones((n,), f32), row, col).reshape(nw, n)

    xs_lo, xs_hi, dinv = pl.pallas_call(
        _mm_first,
        grid=grid,
        in_specs=[rblk(f_in), full(d, f_in),
                  pl.BlockSpec((nw, BLK), lambda i: (0, i))],
        out_specs=(rblk(HALF), rblk(HALF), vec),
        out_shape=(jax.ShapeDtypeStruct((n, HALF), f32),
                   jax.ShapeDtypeStruct((n, HALF), f32),
                   jax.ShapeDtypeStruct((n,), f32)),
    )(x, W0, dp)
    a_lo, a_hi = agg_wide(xs_lo, xs_hi, row, col)

    xs_lo, xs_hi = pl.pallas_call(
        _mm_mid,
        grid=grid,
        in_specs=[rblk(HALF), rblk(HALF), vec, full(1, d), full(d, d)],
        out_specs=(rblk(HALF), rblk(HALF)),
        out_shape=(jax.ShapeDtypeStruct((n, HALF), f32),
                   jax.ShapeDtypeStruct((n, HALF), f32)),
    )(a_lo, a_hi, dinv, b0.reshape(1, d), W1)
    a_lo, a_hi = agg_wide(xs_lo, xs_hi, row, col)

    s = pl.pallas_call(
        _mm_last,
        grid=grid,
        in_specs=[rblk(HALF), rblk(HALF), vec, full(1, d), full(1, d)],
        out_specs=vec,
        out_shape=jax.ShapeDtypeStruct((n,), f32),
    )(a_lo, a_hi, dinv, b1.reshape(1, d), W2)
    sp = agg_scalar(s, row, col).reshape(nw, n)
    return pl.pallas_call(
        _mm_fin,
        grid=grid,
        in_specs=[pl.BlockSpec((nw, BLK), lambda i: (0, i)), vec, vec,
                  pl.BlockSpec((1,), lambda i: (0,))],
        out_specs=vec,
        out_shape=jax.ShapeDtypeStruct((n,), f32),
    )(sp, s, dinv, b2)


# speedup vs baseline: 1.0132x; 1.0132x over previous
import functools

import jax
import jax.numpy as jnp
from jax import lax
from jax.experimental import pallas as pl
from jax.experimental.pallas import tpu as pltpu
from jax.experimental.pallas import tpu_sc as plsc

NC = 2
NS = 16
HALF = 128


def _chunk(epw):
    for k in range(128, 0, -8):
        if epw % k == 0:
            return k
    raise ValueError(epw)


def _row_copy(sid, src, dst, n):
    rpw = (n // NS) // 8 * 8
    tail = n - NS * rpw
    r0 = sid * rpw
    pltpu.sync_copy(src.at[pl.ds(r0, rpw)], dst.at[pl.ds(r0, rpw)])
    if tail:
        @pl.when(sid == NS - 1)
        def _():
            pltpu.sync_copy(src.at[pl.ds(NS * rpw, tail)],
                            dst.at[pl.ds(NS * rpw, tail)])


def _build_agg_wide(n, e):
    assert e % NS == 0 and n % NS == 0
    epw = e // NS
    k = 128
    nchunks = epw // k
    tail = epw - nchunks * k
    assert tail % 8 == 0 and nchunks >= 8
    mesh = plsc.VectorSubcoreMesh(
        core_axis_name="c", subcore_axis_name="s", num_cores=NC, num_subcores=NS
    )

    NBD = 3
    NBC = 3
    NBS = 1
    scratch = ([pltpu.VMEM((k,), jnp.int32)] * NBC
               + [pltpu.VMEM((k,), jnp.int32)] * NBC
               + [pltpu.VMEM((k, HALF), jnp.float32)] * NBD
               + ([pltpu.VMEM((tail,), jnp.int32)] if tail else [])
               + [pltpu.VMEM_SHARED((n, HALF), jnp.float32)]
               + [pltpu.SemaphoreType.DMA] * (2 * NBC + NBD + NBS + (1 if tail else 0)))

    @functools.partial(
        pl.kernel,
        out_type=(
            jax.ShapeDtypeStruct((n, HALF), jnp.float32),
            jax.ShapeDtypeStruct((n, HALF), jnp.float32),
        ),
        mesh=mesh,
        scratch_types=scratch,
    )
    def kern(lo_hbm, hi_hbm, row_hbm, col_hbm, out_lo, out_hi, *scr):
        rows = scr[0:NBC]
        cols = scr[NBC:2 * NBC]
        bufs = scr[2 * NBC:2 * NBC + NBD]
        p = 2 * NBC + NBD
        if tail:
            tcol = scr[p]
            p += 1
        acc_sh = scr[p]
        sems = scr[p + 1:]
        semr = sems[0:NBC]
        semc = sems[NBC:2 * NBC]
        semg = sems[2 * NBC:2 * NBC + NBD]
        sems_ = sems[2 * NBC + NBD:2 * NBC + NBD + NBS]

        cid = lax.axis_index("c")
        sid = lax.axis_index("s")
        ebase = sid * epw

        def run(x_hbm, out_hbm):
            def issue_c(j, rc):
                off = ebase + j * k
                pltpu.async_copy(row_hbm.at[pl.ds(off, k)], rows[rc], semr[rc])
                pltpu.async_copy(col_hbm.at[pl.ds(off, k)], cols[rc], semc[rc])

            def wait_row(rc):
                pltpu.make_async_copy(row_hbm.at[pl.ds(0, k)],
                                      rows[rc], semr[rc]).wait()

            def wait_c(rc):
                pltpu.make_async_copy(col_hbm.at[pl.ds(0, k)],
                                      cols[rc], semc[rc]).wait()

            def issue_g(j, rd):
                pltpu.async_copy(x_hbm.at[rows[rd % NBC]], bufs[rd], semg[rd])

            def wait_g(rd):
                pltpu.make_async_copy(x_hbm.at[rows[rd % NBC]],
                                      bufs[rd], semg[rd]).wait()

            def issue_s(rd, rc, rs):
                pltpu.async_copy(bufs[rd], acc_sh.at[cols[rc]], sems_[rs],
                                 add=True)

            def wait_s(rd, rc, rs):
                pltpu.make_async_copy(bufs[rd], acc_sh.at[cols[rc]],
                                      sems_[rs]).wait()

            def body(j, q, first=False, has_c=True, has_g=True):
                rd, rc, rs = q % NBD, q % NBC, q % NBS
                if not first:
                    wait_s((q - 1) % NBD, (q - 1) % NBC, (q - 1) % NBS)
                if has_c:
                    issue_c(j + 2, (q + 2) % NBC)
                wait_g(rd)
                wait_c(rc)
                issue_s(rd, rc, rs)
                if has_g:
                    wait_row((q + 1) % NBC)
                    issue_g(j + 1, (q + 1) % NBD)

            _row_copy(sid, x_hbm, acc_sh, n)
            plsc.subcore_barrier()

            issue_c(0, 0)
            issue_c(1, 1)
            wait_row(0)
            issue_g(0, 0)
            body(0, 0, first=True)
            body(1, 1)

            steady = ((nchunks - 4) // 3) * 3
            if steady > 0:
                def loop_body(u, _):
                    j0 = 2 + u * 3
                    for q in range(3):
                        body(j0 + q, 2 + q)
                    return 0

                lax.fori_loop(0, steady // 3, loop_body, 0)

            for j in range(2 + steady, nchunks):
                body(j, j, has_c=(j + 2 < nchunks), has_g=(j + 1 < nchunks))
            j = nchunks - 1
            wait_s(j % NBD, j % NBC, j % NBS)

            if tail:
                off = ebase + nchunks * k
                pltpu.sync_copy(row_hbm.at[pl.ds(off, tail)],
                                rows[0].at[pl.ds(0, tail)])
                pltpu.sync_copy(col_hbm.at[pl.ds(off, tail)], tcol)
                tb = bufs[0].at[pl.ds(0, tail)]
                pltpu.async_copy(x_hbm.at[rows[0].at[pl.ds(0, tail)]], tb,
                                 sems[2 * NBC + NBD + NBS]).wait()
                pltpu.sync_copy(tb, acc_sh.at[tcol], add=True)

            plsc.subcore_barrier()
            _row_copy(sid, acc_sh, out_hbm, n)

        @pl.when(cid == 0)
        def _():
            run(lo_hbm, out_lo)

        @pl.when(cid == 1)
        def _():
            run(hi_hbm, out_hi)

    return kern


def _build_agg_scalar(n, e, gather=True):
    nw = NC * NS
    L = 16
    assert n % L == 0
    epw = -(-e // nw)
    epw_pad = -(-epw // L) * L
    assert (epw * (nw - 1)) % 8 == 0 and epw % 8 == 0
    nvec = epw // L
    rem = epw - nvec * L
    mesh = plsc.VectorSubcoreMesh(
        core_axis_name="c", subcore_axis_name="s", num_cores=NC, num_subcores=NS
    )

    scratch = (([pltpu.VMEM((n,), jnp.float32),
                 pltpu.VMEM((epw_pad,), jnp.int32)] if gather else [])
               + [pltpu.VMEM((n,), jnp.float32),
                  pltpu.VMEM((epw_pad,), jnp.int32)])

    @functools.partial(
        pl.kernel,
        out_type=jax.ShapeDtypeStruct((nw * n,), jnp.float32),
        mesh=mesh,
        compiler_params=pltpu.CompilerParams(needs_layout_passes=False),
        scratch_types=scratch,
    )
    def kern(*args):
        if gather:
            tab_hbm, row_hbm, col_hbm, out_hbm, tab_v, row_v, acc_v, col_v = args
        else:
            col_hbm, out_hbm, acc_v, col_v = args
        cid = lax.axis_index("c")
        sid = lax.axis_index("s")
        wid = cid * NS + sid
        ebase = wid * epw

        zeros = jnp.zeros((L,), jnp.float32)

        def zbody(i, _):
            acc_v[pl.ds(i * L, L)] = zeros
            return 0

        lax.fori_loop(0, n // L, zbody, 0)
        if epw_pad > epw:
            col_v[pl.ds(epw_pad - L, L)] = jnp.zeros((L,), jnp.int32)
        pltpu.sync_copy(col_hbm.at[pl.ds(ebase, epw)], col_v.at[pl.ds(0, epw)])
        if gather:
            pltpu.sync_copy(tab_hbm, tab_v)
            if epw_pad > epw:
                row_v[pl.ds(epw_pad - L, L)] = jnp.zeros((L,), jnp.int32)
            pltpu.sync_copy(row_hbm.at[pl.ds(ebase, epw)],
                            row_v.at[pl.ds(0, epw)])

        ones = jnp.ones((L,), jnp.float32)

        def vals_at(j, mask=None):
            if not gather:
                return ones
            return plsc.load_gather(tab_v, [row_v[pl.ds(j * L, L)]], mask=mask)

        def body(j, _):
            cols = col_v[pl.ds(j * L, L)]
            plsc.addupdate_scatter(acc_v, [cols], vals_at(j))
            return 0

        lax.fori_loop(0, nvec, body, 0)
        if rem:
            mask = lax.iota(jnp.int32, L) < rem
            cols = col_v[pl.ds(nvec * L, L)]
            plsc.addupdate_scatter(acc_v, [cols], vals_at(nvec, mask),
                                   mask=mask)
        pltpu.sync_copy(acc_v, out_hbm.at[pl.ds(wid * n, n)])

    return kern


BLK = 2048


def _mm_first(x_ref, w_ref, dp_ref, lo_ref, hi_ref, dinv_ref):
    deg = jnp.sum(dp_ref[...], axis=0) + 1.0
    dinv = lax.rsqrt(deg)
    dinv_ref[...] = dinv
    xl = lax.dot_general(x_ref[...], w_ref[...], (((1,), (1,)), ((), ())),
                         preferred_element_type=jnp.float32)
    xs = xl * dinv[:, None]
    lo_ref[...] = xs[:, :HALF]
    hi_ref[...] = xs[:, HALF:]


def _mm_mid(lo_ref, hi_ref, dinv_ref, b_ref, w_ref, olo_ref, ohi_ref):
    dinv = dinv_ref[...][:, None]
    b = b_ref[...]
    h_lo = jnp.tanh(dinv * lo_ref[...] + b[:, :HALF])
    h_hi = jnp.tanh(dinv * hi_ref[...] + b[:, HALF:])
    w = w_ref[...]
    xl = (lax.dot_general(h_lo, w[:, :HALF], (((1,), (1,)), ((), ())),
                          preferred_element_type=jnp.float32)
          + lax.dot_general(h_hi, w[:, HALF:], (((1,), (1,)), ((), ())),
                            preferred_element_type=jnp.float32))
    xs = xl * dinv
    olo_ref[...] = xs[:, :HALF]
    ohi_ref[...] = xs[:, HALF:]


def _mm_last(lo_ref, hi_ref, dinv_ref, b_ref, w2_ref, s_ref):
    dinv = dinv_ref[...][:, None]
    b = b_ref[...]
    h_lo = jnp.tanh(dinv * lo_ref[...] + b[:, :HALF])
    h_hi = jnp.tanh(dinv * hi_ref[...] + b[:, HALF:])
    w2 = w2_ref[...]
    s = (lax.dot_general(h_lo, w2[:, :HALF], (((1,), (1,)), ((), ())),
                         preferred_element_type=jnp.float32)
         + lax.dot_general(h_hi, w2[:, HALF:], (((1,), (1,)), ((), ())),
                           preferred_element_type=jnp.float32))
    s_ref[...] = (s * dinv)[:, 0]


def _mm_fin(sp_ref, s_ref, dinv_ref, b2_ref, o_ref):
    agg = jnp.sum(sp_ref[...], axis=0) + s_ref[...]
    z = dinv_ref[...] * agg + b2_ref[...]
    o_ref[...] = 1.0 / (1.0 + jnp.exp(-z))


def kernel(x, edge_index, W0, b0, W1, b1, W2, b2):
    n, f_in = x.shape
    d = W0.shape[0]
    e = edge_index.shape[1]
    nw = NC * NS
    row = edge_index[0]
    col = edge_index[1]

    agg_wide = _build_agg_wide(n, e)
    agg_scalar = _build_agg_scalar(n, e)
    agg_count = _build_agg_scalar(n, e, gather=False)
    f32 = jnp.float32
    grid = (pl.cdiv(n, BLK),)

    def rblk(w):
        return pl.BlockSpec((BLK, w), lambda i: (i, 0))

    def full(a, b):
        return pl.BlockSpec((a, b), lambda i: (0, 0))

    vec = pl.BlockSpec((BLK,), lambda i: (i,))

    dp = agg_count(col).reshape(nw, n)

    xs_lo, xs_hi, dinv = pl.pallas_call(
        _mm_first,
        grid=grid,
        in_specs=[rblk(f_in), full(d, f_in),
                  pl.BlockSpec((nw, BLK), lambda i: (0, i))],
        out_specs=(rblk(HALF), rblk(HALF), vec),
        out_shape=(jax.ShapeDtypeStruct((n, HALF), f32),
                   jax.ShapeDtypeStruct((n, HALF), f32),
                   jax.ShapeDtypeStruct((n,), f32)),
    )(x, W0, dp)
    a_lo, a_hi = agg_wide(xs_lo, xs_hi, row, col)

    xs_lo, xs_hi = pl.pallas_call(
        _mm_mid,
        grid=grid,
        in_specs=[rblk(HALF), rblk(HALF), vec, full(1, d), full(d, d)],
        out_specs=(rblk(HALF), rblk(HALF)),
        out_shape=(jax.ShapeDtypeStruct((n, HALF), f32),
                   jax.ShapeDtypeStruct((n, HALF), f32)),
    )(a_lo, a_hi, dinv, b0.reshape(1, d), W1)
    a_lo, a_hi = agg_wide(xs_lo, xs_hi, row, col)

    s = pl.pallas_call(
        _mm_last,
        grid=grid,
        in_specs=[rblk(HALF), rblk(HALF), vec, full(1, d), full(1, d)],
        out_specs=vec,
        out_shape=jax.ShapeDtypeStruct((n,), f32),
    )(a_lo, a_hi, dinv, b1.reshape(1, d), W2)
    sp = agg_scalar(s, row, col).reshape(nw, n)
    return pl.pallas_call(
        _mm_fin,
        grid=grid,
        in_specs=[pl.BlockSpec((nw, BLK), lambda i: (0, i)), vec, vec,
                  pl.BlockSpec((1,), lambda i: (0,))],
        out_specs=vec,
        out_shape=jax.ShapeDtypeStruct((n,), f32),
    )(sp, s, dinv, b2)


# speedup vs baseline: 1.0230x; 1.0096x over previous
import functools

import jax
import jax.numpy as jnp
from jax import lax
from jax.experimental import pallas as pl
from jax.experimental.pallas import tpu as pltpu
from jax.experimental.pallas import tpu_sc as plsc

NC = 2
NS = 16
HALF = 128


def _row_copy(sid, src, dst, n):
    rpw = (n // NS) // 8 * 8
    tail = n - NS * rpw
    r0 = sid * rpw
    pltpu.sync_copy(src.at[pl.ds(r0, rpw)], dst.at[pl.ds(r0, rpw)])
    if tail:
        @pl.when(sid == NS - 1)
        def _():
            pltpu.sync_copy(src.at[pl.ds(NS * rpw, tail)],
                            dst.at[pl.ds(NS * rpw, tail)])


def _build_agg_wide(n, e):
    assert e % NS == 0 and n % NS == 0
    epw = e // NS
    k = 128
    nchunks = epw // k
    tail = epw - nchunks * k
    assert tail % 8 == 0 and nchunks >= 8
    mesh = plsc.VectorSubcoreMesh(
        core_axis_name="c", subcore_axis_name="s", num_cores=NC, num_subcores=NS
    )

    NBD = 3
    NBC = 3
    NBS = 1
    scratch = ([pltpu.VMEM((k,), jnp.int32)] * NBC
               + [pltpu.VMEM((k,), jnp.int32)] * NBC
               + [pltpu.VMEM((k, HALF), jnp.float32)] * NBD
               + ([pltpu.VMEM((tail,), jnp.int32)] if tail else [])
               + [pltpu.VMEM_SHARED((n, HALF), jnp.float32)]
               + [pltpu.SemaphoreType.DMA] * (2 * NBC + NBD + NBS + (1 if tail else 0)))

    @functools.partial(
        pl.kernel,
        out_type=(
            jax.ShapeDtypeStruct((n, HALF), jnp.float32),
            jax.ShapeDtypeStruct((n, HALF), jnp.float32),
        ),
        mesh=mesh,
        scratch_types=scratch,
    )
    def kern(lo_hbm, hi_hbm, row_hbm, col_hbm, out_lo, out_hi, *scr):
        rows = scr[0:NBC]
        cols = scr[NBC:2 * NBC]
        bufs = scr[2 * NBC:2 * NBC + NBD]
        p = 2 * NBC + NBD
        if tail:
            tcol = scr[p]
            p += 1
        acc_sh = scr[p]
        sems = scr[p + 1:]
        semr = sems[0:NBC]
        semc = sems[NBC:2 * NBC]
        semg = sems[2 * NBC:2 * NBC + NBD]
        sems_ = sems[2 * NBC + NBD:2 * NBC + NBD + NBS]

        cid = lax.axis_index("c")
        sid = lax.axis_index("s")
        ebase = sid * epw

        def run(x_hbm, out_hbm):
            def issue_c(j, rc):
                off = ebase + j * k
                pltpu.async_copy(row_hbm.at[pl.ds(off, k)], rows[rc], semr[rc])
                pltpu.async_copy(col_hbm.at[pl.ds(off, k)], cols[rc], semc[rc])

            def wait_row(rc):
                pltpu.make_async_copy(row_hbm.at[pl.ds(0, k)],
                                      rows[rc], semr[rc]).wait()

            def wait_c(rc):
                pltpu.make_async_copy(col_hbm.at[pl.ds(0, k)],
                                      cols[rc], semc[rc]).wait()

            def issue_g(j, rd):
                pltpu.async_copy(x_hbm.at[rows[rd % NBC]], bufs[rd], semg[rd])

            def wait_g(rd):
                pltpu.make_async_copy(x_hbm.at[rows[rd % NBC]],
                                      bufs[rd], semg[rd]).wait()

            def issue_s(rd, rc, rs):
                pltpu.async_copy(bufs[rd], acc_sh.at[cols[rc]], sems_[rs],
                                 add=True)

            def wait_s(rd, rc, rs):
                pltpu.make_async_copy(bufs[rd], acc_sh.at[cols[rc]],
                                      sems_[rs]).wait()

            def body(j, q, first=False, has_c=True, has_g=True):
                rd, rc, rs = q % NBD, q % NBC, q % NBS
                if not first:
                    wait_s((q - 1) % NBD, (q - 1) % NBC, (q - 1) % NBS)
                if has_c:
                    issue_c(j + 2, (q + 2) % NBC)
                wait_g(rd)
                wait_c(rc)
                issue_s(rd, rc, rs)
                if has_g:
                    wait_row((q + 1) % NBC)
                    issue_g(j + 1, (q + 1) % NBD)

            issue_c(0, 0)
            issue_c(1, 1)
            _row_copy(sid, x_hbm, acc_sh, n)
            wait_row(0)
            issue_g(0, 0)
            plsc.subcore_barrier()
            body(0, 0, first=True)
            body(1, 1)

            steady = ((nchunks - 4) // 3) * 3
            if steady > 0:
                def loop_body(u, _):
                    j0 = 2 + u * 3
                    for q in range(3):
                        body(j0 + q, 2 + q)
                    return 0

                lax.fori_loop(0, steady // 3, loop_body, 0)

            for j in range(2 + steady, nchunks):
                body(j, j, has_c=(j + 2 < nchunks), has_g=(j + 1 < nchunks))
            j = nchunks - 1
            wait_s(j % NBD, j % NBC, j % NBS)

            if tail:
                off = ebase + nchunks * k
                pltpu.sync_copy(row_hbm.at[pl.ds(off, tail)],
                                rows[0].at[pl.ds(0, tail)])
                pltpu.sync_copy(col_hbm.at[pl.ds(off, tail)], tcol)
                tb = bufs[0].at[pl.ds(0, tail)]
                pltpu.async_copy(x_hbm.at[rows[0].at[pl.ds(0, tail)]], tb,
                                 sems[2 * NBC + NBD + NBS]).wait()
                pltpu.sync_copy(tb, acc_sh.at[tcol], add=True)

            plsc.subcore_barrier()
            _row_copy(sid, acc_sh, out_hbm, n)

        @pl.when(cid == 0)
        def _():
            run(lo_hbm, out_lo)

        @pl.when(cid == 1)
        def _():
            run(hi_hbm, out_hi)

    return kern


def _build_agg_scalar(n, e, gather=True):
    nw = NC * NS
    L = 16
    assert n % L == 0
    epw = -(-e // nw)
    epw_pad = -(-epw // L) * L
    assert (epw * (nw - 1)) % 8 == 0 and epw % 8 == 0
    nvec = epw // L
    rem = epw - nvec * L
    mesh = plsc.VectorSubcoreMesh(
        core_axis_name="c", subcore_axis_name="s", num_cores=NC, num_subcores=NS
    )

    scratch = (([pltpu.VMEM((n,), jnp.float32),
                 pltpu.VMEM((epw_pad,), jnp.int32)] if gather else [])
               + [pltpu.VMEM((n,), jnp.float32),
                  pltpu.VMEM((epw_pad,), jnp.int32)])

    @functools.partial(
        pl.kernel,
        out_type=jax.ShapeDtypeStruct((nw * n,), jnp.float32),
        mesh=mesh,
        compiler_params=pltpu.CompilerParams(needs_layout_passes=False),
        scratch_types=scratch,
    )
    def kern(*args):
        if gather:
            tab_hbm, row_hbm, col_hbm, out_hbm, tab_v, row_v, acc_v, col_v = args
        else:
            col_hbm, out_hbm, acc_v, col_v = args
        cid = lax.axis_index("c")
        sid = lax.axis_index("s")
        wid = cid * NS + sid
        ebase = wid * epw

        zeros = jnp.zeros((L,), jnp.float32)

        def zbody(i, _):
            acc_v[pl.ds(i * L, L)] = zeros
            return 0

        lax.fori_loop(0, n // L, zbody, 0)
        if epw_pad > epw:
            col_v[pl.ds(epw_pad - L, L)] = jnp.zeros((L,), jnp.int32)
        pltpu.sync_copy(col_hbm.at[pl.ds(ebase, epw)], col_v.at[pl.ds(0, epw)])
        if gather:
            pltpu.sync_copy(tab_hbm, tab_v)
            if epw_pad > epw:
                row_v[pl.ds(epw_pad - L, L)] = jnp.zeros((L,), jnp.int32)
            pltpu.sync_copy(row_hbm.at[pl.ds(ebase, epw)],
                            row_v.at[pl.ds(0, epw)])

        ones = jnp.ones((L,), jnp.float32)

        def vals_at(j, mask=None):
            if not gather:
                return ones
            return plsc.load_gather(tab_v, [row_v[pl.ds(j * L, L)]], mask=mask)

        def body(j, _):
            cols = col_v[pl.ds(j * L, L)]
            plsc.addupdate_scatter(acc_v, [cols], vals_at(j))
            return 0

        lax.fori_loop(0, nvec, body, 0)
        if rem:
            mask = lax.iota(jnp.int32, L) < rem
            cols = col_v[pl.ds(nvec * L, L)]
            plsc.addupdate_scatter(acc_v, [cols], vals_at(nvec, mask),
                                   mask=mask)
        pltpu.sync_copy(acc_v, out_hbm.at[pl.ds(wid * n, n)])

    return kern


BLK = 2048


def _mm_first(x_ref, w_ref, dp_ref, lo_ref, hi_ref, dinv_ref):
    deg = jnp.sum(dp_ref[...], axis=0) + 1.0
    dinv = lax.rsqrt(deg)
    dinv_ref[...] = dinv
    xl = lax.dot_general(x_ref[...], w_ref[...], (((1,), (1,)), ((), ())),
                         preferred_element_type=jnp.float32)
    xs = xl * dinv[:, None]
    lo_ref[...] = xs[:, :HALF]
    hi_ref[...] = xs[:, HALF:]


def _mm_mid(lo_ref, hi_ref, dinv_ref, b_ref, w_ref, olo_ref, ohi_ref):
    dinv = dinv_ref[...][:, None]
    b = b_ref[...]
    h_lo = jnp.tanh(dinv * lo_ref[...] + b[:, :HALF])
    h_hi = jnp.tanh(dinv * hi_ref[...] + b[:, HALF:])
    w = w_ref[...]
    xl = (lax.dot_general(h_lo, w[:, :HALF], (((1,), (1,)), ((), ())),
                          preferred_element_type=jnp.float32)
          + lax.dot_general(h_hi, w[:, HALF:], (((1,), (1,)), ((), ())),
                            preferred_element_type=jnp.float32))
    xs = xl * dinv
    olo_ref[...] = xs[:, :HALF]
    ohi_ref[...] = xs[:, HALF:]


def _mm_last(lo_ref, hi_ref, dinv_ref, b_ref, w2_ref, s_ref):
    dinv = dinv_ref[...][:, None]
    b = b_ref[...]
    h_lo = jnp.tanh(dinv * lo_ref[...] + b[:, :HALF])
    h_hi = jnp.tanh(dinv * hi_ref[...] + b[:, HALF:])
    w2 = w2_ref[...]
    s = (lax.dot_general(h_lo, w2[:, :HALF], (((1,), (1,)), ((), ())),
                         preferred_element_type=jnp.float32)
         + lax.dot_general(h_hi, w2[:, HALF:], (((1,), (1,)), ((), ())),
                           preferred_element_type=jnp.float32))
    s_ref[...] = (s * dinv)[:, 0]


def _mm_fin(sp_ref, s_ref, dinv_ref, b2_ref, o_ref):
    agg = jnp.sum(sp_ref[...], axis=0) + s_ref[...]
    z = dinv_ref[...] * agg + b2_ref[...]
    o_ref[...] = 1.0 / (1.0 + jnp.exp(-z))


def kernel(x, edge_index, W0, b0, W1, b1, W2, b2):
    n, f_in = x.shape
    d = W0.shape[0]
    e = edge_index.shape[1]
    nw = NC * NS
    row = edge_index[0]
    col = edge_index[1]

    agg_wide = _build_agg_wide(n, e)
    agg_scalar = _build_agg_scalar(n, e)
    agg_count = _build_agg_scalar(n, e, gather=False)
    f32 = jnp.float32
    grid = (pl.cdiv(n, BLK),)

    def rblk(w):
        return pl.BlockSpec((BLK, w), lambda i: (i, 0))

    def full(a, b):
        return pl.BlockSpec((a, b), lambda i: (0, 0))

    vec = pl.BlockSpec((BLK,), lambda i: (i,))

    dp = agg_count(col).reshape(nw, n)

    xs_lo, xs_hi, dinv = pl.pallas_call(
        _mm_first,
        grid=grid,
        in_specs=[rblk(f_in), full(d, f_in),
                  pl.BlockSpec((nw, BLK), lambda i: (0, i))],
        out_specs=(rblk(HALF), rblk(HALF), vec),
        out_shape=(jax.ShapeDtypeStruct((n, HALF), f32),
                   jax.ShapeDtypeStruct((n, HALF), f32),
                   jax.ShapeDtypeStruct((n,), f32)),
    )(x, W0, dp)
    a_lo, a_hi = agg_wide(xs_lo, xs_hi, row, col)

    xs_lo, xs_hi = pl.pallas_call(
        _mm_mid,
        grid=grid,
        in_specs=[rblk(HALF), rblk(HALF), vec, full(1, d), full(d, d)],
        out_specs=(rblk(HALF), rblk(HALF)),
        out_shape=(jax.ShapeDtypeStruct((n, HALF), f32),
                   jax.ShapeDtypeStruct((n, HALF), f32)),
    )(a_lo, a_hi, dinv, b0.reshape(1, d), W1)
    a_lo, a_hi = agg_wide(xs_lo, xs_hi, row, col)

    s = pl.pallas_call(
        _mm_last,
        grid=grid,
        in_specs=[rblk(HALF), rblk(HALF), vec, full(1, d), full(1, d)],
        out_specs=vec,
        out_shape=jax.ShapeDtypeStruct((n,), f32),
    )(a_lo, a_hi, dinv, b1.reshape(1, d), W2)
    sp = agg_scalar(s, row, col).reshape(nw, n)
    return pl.pallas_call(
        _mm_fin,
        grid=grid,
        in_specs=[pl.BlockSpec((nw, BLK), lambda i: (0, i)), vec, vec,
                  pl.BlockSpec((1,), lambda i: (0,))],
        out_specs=vec,
        out_shape=jax.ShapeDtypeStruct((n,), f32),
    )(sp, s, dinv, b2)
